# baseline (speedup 1.0000x reference)
import functools

import jax
import jax.numpy as jnp
from jax import lax
from jax.experimental import pallas as pl
from jax.experimental.pallas import tpu as pltpu
from jax.experimental.pallas import tpu_sc as plsc

VOCAB = 100000
HIDDEN = 1024
MAX_POS = 2048
B, S = 4, 2048
EPS = 1e-12

NW = 32
SPW = S // NW
TOK_PER_W = B * SPW
CHUNK = 16
NCHUNK = TOK_PER_W // CHUNK
NPAIR = NCHUNK // 2
LANES = 16
JV = HIDDEN // LANES


def _lane_sum(x):
    iota = lax.iota(jnp.int32, LANES)
    for shift in (1, 2, 4, 8):
        idx = (iota + shift) & (LANES - 1)
        x = x + _perm(x, idx)
    return x


_DNUMS = lax.GatherDimensionNumbers(
    offset_dims=(), collapsed_slice_dims=(0,), start_index_map=(0,))


def _perm(x, idx):
    return lax.gather(x, idx[:, None], _DNUMS, (1,),
                      mode=lax.GatherScatterMode.PROMISE_IN_BOUNDS)


def _rsqrt_vec(x):
    y = 1.0 / (x * 8.0 + 0.02)
    half = x * 0.5
    for _ in range(6):
        y = y * (1.5 - half * y * y)
    return y


def _make_kernel():
    mesh = plsc.VectorSubcoreMesh(core_axis_name="c", subcore_axis_name="s")

    set_scratch = [
        pltpu.VMEM((CHUNK, HIDDEN), jnp.float32),
        pltpu.VMEM((CHUNK, LANES), jnp.float32),
        pltpu.VMEM((CHUNK, LANES), jnp.float32),
        pltpu.SemaphoreType.DMA,
        pltpu.SemaphoreType.DMA,
    ]

    @functools.partial(
        pl.kernel,
        mesh=mesh,
        out_type=jax.ShapeDtypeStruct((B * S, HIDDEN), jnp.float32),
        scratch_types=[
            pltpu.VMEM((TOK_PER_W,), jnp.int32),
            pltpu.VMEM((TOK_PER_W,), jnp.int32),
            pltpu.VMEM((SPW, HIDDEN), jnp.float32),
            pltpu.VMEM((HIDDEN,), jnp.float32),
            pltpu.VMEM((HIDDEN,), jnp.float32),
            pltpu.VMEM((HIDDEN,), jnp.float32),
        ] + set_scratch + set_scratch,
    )
    def k(ids_hbm, tt_hbm, tok_hbm, pos_hbm, dseg_hbm, gamma_hbm, beta_hbm,
          out_hbm, idx_all, tti_all, pos_v, gamma_v, beta_v, d_v,
          a_buf, a_mean, a_rstd, a_sg, a_so,
          b_buf, b_mean, b_rstd, b_sg, b_so):
        wid = lax.axis_index("c") * 16 + lax.axis_index("s")
        w64 = wid * SPW

        sets = ((a_buf, a_mean, a_rstd, a_sg, a_so),
                (b_buf, b_mean, b_rstd, b_sg, b_so))

        pltpu.sync_copy(gamma_hbm, gamma_v)
        pltpu.sync_copy(beta_hbm, beta_v)
        pltpu.sync_copy(dseg_hbm, d_v)
        pltpu.sync_copy(pos_hbm.at[pl.ds(w64, SPW)], pos_v)
        for b in range(B):
            pltpu.sync_copy(ids_hbm.at[pl.ds(b * S + w64, SPW)],
                            idx_all.at[pl.ds(b * SPW, SPW)])
            pltpu.sync_copy(tt_hbm.at[pl.ds(b * S + w64, SPW)],
                            tti_all.at[pl.ds(b * SPW, SPW)])

        def chunk_coords(kk):
            b = kk & 3
            sb = kk >> 2
            co = b * SPW + sb * CHUNK
            g0 = b * S + w64 + sb * CHUNK
            return co, g0, sb * CHUNK

        def issue_gathers(st, kk):
            co, _, _ = chunk_coords(kk)
            idx_vec = idx_all[pl.ds(co, CHUNK)]
            pltpu.async_copy(tok_hbm.at[idx_vec], st[0], st[3])

        def wait_gathers(st):
            zidx = jnp.zeros((CHUNK,), jnp.int32)
            pltpu.make_async_copy(tok_hbm.at[zidx], st[0], st[3]).wait()

        def issue_out(st, kk):
            _, g0, _ = chunk_coords(kk)
            pltpu.async_copy(st[0], out_hbm.at[pl.ds(g0, CHUNK)], st[4])

        def wait_out(st):
            pltpu.make_async_copy(
                st[0], out_hbm.at[pl.ds(0, CHUNK)], st[4]).wait()

        def compute(st, kk):
            buf, mean_r, rstd_r = st[0], st[1], st[2]
            co, _, srow0 = chunk_coords(kk)
            ttv = tti_all[pl.ds(co, CHUNK)].astype(jnp.float32)

            @plsc.parallel_loop(0, CHUNK, unroll=2)
            def stats(t):
                ttf = _perm(ttv, jnp.full((LANES,), t, jnp.int32))
                srow = srow0 + t
                a0 = jnp.zeros((LANES,), jnp.float32)
                q0 = jnp.zeros((LANES,), jnp.float32)
                a1 = jnp.zeros((LANES,), jnp.float32)
                q1 = jnp.zeros((LANES,), jnp.float32)
                for j in range(0, JV, 2):
                    sl0 = pl.ds(j * LANES, LANES)
                    sl1 = pl.ds((j + 1) * LANES, LANES)
                    v0 = buf[t, sl0] + pos_v[srow, sl0] + ttf * d_v[sl0]
                    v1 = buf[t, sl1] + pos_v[srow, sl1] + ttf * d_v[sl1]
                    buf[t, sl0] = v0
                    buf[t, sl1] = v1
                    a0 = a0 + v0
                    q0 = q0 + v0 * v0
                    a1 = a1 + v1
                    q1 = q1 + v1 * v1
                mean_r[t, :] = a0 + a1
                rstd_r[t, :] = q0 + q1

            @plsc.parallel_loop(0, CHUNK, unroll=4)
            def finish(t):
                ssum_v = _lane_sum(mean_r[t, :])
                qsum_v = _lane_sum(rstd_r[t, :])
                mean_v = ssum_v * (1.0 / HIDDEN)
                var_v = qsum_v * (1.0 / HIDDEN) - mean_v * mean_v
                mean_r[t, :] = mean_v
                rstd_r[t, :] = _rsqrt_vec(var_v + EPS)

            for js in range(4):
                jbase = js * (JV // 4)
                gv = [gamma_v[pl.ds((jbase + jj) * LANES, LANES)]
                      for jj in range(JV // 4)]
                bv = [beta_v[pl.ds((jbase + jj) * LANES, LANES)]
                      for jj in range(JV // 4)]

                @plsc.parallel_loop(0, CHUNK, unroll=1)
                def apply(t):
                    mean_v = mean_r[t, :]
                    rstd_v = rstd_r[t, :]
                    for jj in range(JV // 4):
                        sl = pl.ds((jbase + jj) * LANES, LANES)
                        y = (buf[t, sl] - mean_v) * rstd_v * gv[jj] + bv[jj]
                        buf[t, sl] = y

        issue_gathers(sets[0], 0)

        def pair_body(kp, _c):
            k0 = 2 * kp
            s0_, s1_ = sets

            @pl.when(kp > 0)
            def _():
                wait_out(s1_)
            issue_gathers(s1_, k0 + 1)

            wait_gathers(s0_)
            compute(s0_, k0)
            issue_out(s0_, k0)

            wait_gathers(s1_)
            compute(s1_, k0 + 1)
            issue_out(s1_, k0 + 1)

            wait_out(s0_)

            @pl.when(kp < NPAIR - 1)
            def _():
                issue_gathers(s0_, k0 + 2)
            return _c

        lax.fori_loop(0, NPAIR, pair_body, 0)
        wait_out(sets[1])

    return k


_kernel_call = _make_kernel()


def kernel(input_ids, token_type_ids, token_table, pos_table, seg_table,
           gamma, beta):
    ids = input_ids.reshape(-1).astype(jnp.int32)
    tt = token_type_ids.reshape(-1).astype(jnp.int32)
    pos_eff = pos_table + seg_table[0][None, :]
    dseg = seg_table[1] - seg_table[0]
    out = _kernel_call(ids, tt, token_table, pos_eff, dseg, gamma, beta)
    return out.reshape(B, S, HIDDEN)


# speedup vs baseline: 1.0586x; 1.0586x over previous
import functools

import jax
import jax.numpy as jnp
from jax import lax
from jax.experimental import pallas as pl
from jax.experimental.pallas import tpu as pltpu
from jax.experimental.pallas import tpu_sc as plsc

VOCAB = 100000
HIDDEN = 1024
MAX_POS = 2048
B, S = 4, 2048
EPS = 1e-12

NW = 32
SPW = S // NW
TOK_PER_W = B * SPW
CHUNK = 16
NCHUNK = TOK_PER_W // CHUNK
NPAIR = NCHUNK // 2
LANES = 16
JV = HIDDEN // LANES


def _lane_sum(x):
    iota = lax.iota(jnp.int32, LANES)
    for shift in (1, 2, 4, 8):
        idx = (iota + shift) & (LANES - 1)
        x = x + _perm(x, idx)
    return x


_DNUMS = lax.GatherDimensionNumbers(
    offset_dims=(), collapsed_slice_dims=(0,), start_index_map=(0,))


def _perm(x, idx):
    return lax.gather(x, idx[:, None], _DNUMS, (1,),
                      mode=lax.GatherScatterMode.PROMISE_IN_BOUNDS)


def _rsqrt_vec(x):
    y = 1.0 / (x * 8.0 + 0.02)
    half = x * 0.5
    for _ in range(6):
        y = y * (1.5 - half * y * y)
    return y


def _make_kernel():
    mesh = plsc.VectorSubcoreMesh(core_axis_name="c", subcore_axis_name="s")

    set_scratch = [
        pltpu.VMEM((CHUNK, HIDDEN), jnp.float32),
        pltpu.VMEM((CHUNK, LANES), jnp.float32),
        pltpu.VMEM((CHUNK, LANES), jnp.float32),
        pltpu.SemaphoreType.DMA,
        pltpu.SemaphoreType.DMA,
    ]

    @functools.partial(
        pl.kernel,
        mesh=mesh,
        out_type=jax.ShapeDtypeStruct((B * S, HIDDEN), jnp.float32),
        scratch_types=[
            pltpu.VMEM((TOK_PER_W,), jnp.int32),
            pltpu.VMEM((TOK_PER_W,), jnp.int32),
            pltpu.VMEM((SPW, HIDDEN), jnp.float32),
            pltpu.VMEM((HIDDEN,), jnp.float32),
            pltpu.VMEM((HIDDEN,), jnp.float32),
            pltpu.VMEM((HIDDEN,), jnp.float32),
            pltpu.SemaphoreType.DMA,
        ] + set_scratch + set_scratch,
    )
    def k(ids_hbm, tt_hbm, tok_hbm, pos_hbm, dseg_hbm, gamma_hbm, beta_hbm,
          out_hbm, idx_all, tti_all, pos_v, gamma_v, beta_v, d_v, s_st,
          a_buf, a_mean, a_rstd, a_sg, a_so,
          b_buf, b_mean, b_rstd, b_sg, b_so):
        wid = lax.axis_index("c") * 16 + lax.axis_index("s")
        w64 = wid * SPW

        sets = ((a_buf, a_mean, a_rstd, a_sg, a_so),
                (b_buf, b_mean, b_rstd, b_sg, b_so))

        id_copies = []
        for b in range(B):
            id_copies.append(pltpu.async_copy(
                ids_hbm.at[pl.ds(b * S + w64, SPW)],
                idx_all.at[pl.ds(b * SPW, SPW)], a_so))
        rest = [
            pltpu.async_copy(gamma_hbm, gamma_v, s_st),
            pltpu.async_copy(beta_hbm, beta_v, s_st),
            pltpu.async_copy(dseg_hbm, d_v, s_st),
            pltpu.async_copy(pos_hbm.at[pl.ds(w64, SPW)], pos_v, s_st),
        ]
        for b in range(B):
            rest.append(pltpu.async_copy(
                tt_hbm.at[pl.ds(b * S + w64, SPW)],
                tti_all.at[pl.ds(b * SPW, SPW)], s_st))

        def chunk_coords(kk):
            b = kk & 3
            sb = kk >> 2
            co = b * SPW + sb * CHUNK
            g0 = b * S + w64 + sb * CHUNK
            return co, g0, sb * CHUNK

        def issue_gathers(st, kk):
            co, _, _ = chunk_coords(kk)
            idx_vec = idx_all[pl.ds(co, CHUNK)]
            pltpu.async_copy(tok_hbm.at[idx_vec], st[0], st[3])

        def wait_gathers(st):
            zidx = jnp.zeros((CHUNK,), jnp.int32)
            pltpu.make_async_copy(tok_hbm.at[zidx], st[0], st[3]).wait()

        def issue_out(st, kk):
            _, g0, _ = chunk_coords(kk)
            pltpu.async_copy(st[0], out_hbm.at[pl.ds(g0, CHUNK)], st[4])

        def wait_out(st):
            pltpu.make_async_copy(
                st[0], out_hbm.at[pl.ds(0, CHUNK)], st[4]).wait()

        def compute(st, kk):
            buf, mean_r, rstd_r = st[0], st[1], st[2]
            co, _, srow0 = chunk_coords(kk)
            ttv = tti_all[pl.ds(co, CHUNK)].astype(jnp.float32)

            @plsc.parallel_loop(0, CHUNK, unroll=2)
            def stats(t):
                ttf = _perm(ttv, jnp.full((LANES,), t, jnp.int32))
                srow = srow0 + t
                a0 = jnp.zeros((LANES,), jnp.float32)
                q0 = jnp.zeros((LANES,), jnp.float32)
                a1 = jnp.zeros((LANES,), jnp.float32)
                q1 = jnp.zeros((LANES,), jnp.float32)
                for j in range(0, JV, 2):
                    sl0 = pl.ds(j * LANES, LANES)
                    sl1 = pl.ds((j + 1) * LANES, LANES)
                    v0 = buf[t, sl0] + pos_v[srow, sl0] + ttf * d_v[sl0]
                    v1 = buf[t, sl1] + pos_v[srow, sl1] + ttf * d_v[sl1]
                    buf[t, sl0] = v0
                    buf[t, sl1] = v1
                    a0 = a0 + v0
                    q0 = q0 + v0 * v0
                    a1 = a1 + v1
                    q1 = q1 + v1 * v1
                ssum_v = _lane_sum(a0 + a1)
                qsum_v = _lane_sum(q0 + q1)
                mean_v = ssum_v * (1.0 / HIDDEN)
                var_v = qsum_v * (1.0 / HIDDEN) - mean_v * mean_v
                mean_r[t, :] = mean_v
                rstd_r[t, :] = _rsqrt_vec(var_v + EPS)

            for js in range(4):
                jbase = js * (JV // 4)
                gv = [gamma_v[pl.ds((jbase + jj) * LANES, LANES)]
                      for jj in range(JV // 4)]
                bv = [beta_v[pl.ds((jbase + jj) * LANES, LANES)]
                      for jj in range(JV // 4)]

                @plsc.parallel_loop(0, CHUNK, unroll=1)
                def apply(t):
                    mean_v = mean_r[t, :]
                    rstd_v = rstd_r[t, :]
                    for jj in range(JV // 4):
                        sl = pl.ds((jbase + jj) * LANES, LANES)
                        y = (buf[t, sl] - mean_v) * rstd_v * gv[jj] + bv[jj]
                        buf[t, sl] = y

        for cp in id_copies:
            cp.wait()
        issue_gathers(sets[0], 0)
        for cp in rest:
            cp.wait()

        def pair_body(kp, _c):
            k0 = 2 * kp
            s0_, s1_ = sets

            @pl.when(kp > 0)
            def _():
                wait_out(s1_)
            issue_gathers(s1_, k0 + 1)

            wait_gathers(s0_)
            compute(s0_, k0)
            issue_out(s0_, k0)

            wait_gathers(s1_)
            compute(s1_, k0 + 1)
            issue_out(s1_, k0 + 1)

            wait_out(s0_)

            @pl.when(kp < NPAIR - 1)
            def _():
                issue_gathers(s0_, k0 + 2)
            return _c

        lax.fori_loop(0, NPAIR, pair_body, 0)
        wait_out(sets[1])

    return k


_kernel_call = _make_kernel()


def kernel(input_ids, token_type_ids, token_table, pos_table, seg_table,
           gamma, beta):
    ids = input_ids.reshape(-1).astype(jnp.int32)
    tt = token_type_ids.reshape(-1).astype(jnp.int32)
    pos_eff = pos_table + seg_table[0][None, :]
    dseg = seg_table[1] - seg_table[0]
    out = _kernel_call(ids, tt, token_table, pos_eff, dseg, gamma, beta)
    return out.reshape(B, S, HIDDEN)
